# baseline (speedup 1.0000x reference)
import functools

import jax
import jax.numpy as jnp
from jax import lax
from jax.experimental import pallas as pl
from jax.experimental.pallas import tpu as pltpu
from jax.experimental.pallas import tpu_sc as plsc

VOCAB_SIZE = 1000000
EMBED_DIM = 64
BATCH = 4096
HIST_LEN = 200

_NC = 2
_NS = 16
_NW = _NC * _NS
_CHUNK = 128
_N_IDX = BATCH * HIST_LEN
_PER_W = _N_IDX // _NW
_N_CHUNKS = _PER_W // _CHUNK

_BLKV = 2048
_GRID = (VOCAB_SIZE + _BLKV - 1) // _BLKV


def _prep_body(t_ref, o_ref):
    ident = jnp.eye(EMBED_DIM, dtype=jnp.float32)
    at = lax.dot_general(
        t_ref[...], ident, (((0,), (0,)), ((), ())),
        preferred_element_type=jnp.float32)
    a = at.reshape(_BLKV // 2, 2, EMBED_DIM)
    o_ref[...] = jnp.concatenate([a[:, 0, :], a[:, 1, :]], axis=-1)


def _prep_tc(table_t):
    return pl.pallas_call(
        _prep_body,
        grid=(_GRID,),
        in_specs=[pl.BlockSpec((EMBED_DIM, _BLKV), lambda i: (0, i))],
        out_specs=pl.BlockSpec((_BLKV // 2, 2 * EMBED_DIM),
                               lambda i: (i, 0)),
        out_shape=jax.ShapeDtypeStruct(
            (VOCAB_SIZE // 2, 2 * EMBED_DIM), jnp.float32),
    )(table_t)


def _emb_kernel(idx_hbm, table_hbm, out_hbm, idx_v, rows0, rows1, sem0, sem1):
    wid = lax.axis_index("s") * _NC + lax.axis_index("c")
    base = wid * _PER_W

    pltpu.sync_copy(idx_hbm.at[wid], idx_v)

    pltpu.async_copy(table_hbm.at[idx_v.at[0]], rows0, sem0)
    pltpu.async_copy(table_hbm.at[idx_v.at[1]], rows1, sem1)

    def body(t, carry):
        j0 = 2 * t

        def step(rows_b, sem_b, j):
            pltpu.make_async_copy(
                table_hbm.at[idx_v.at[j]], rows_b, sem_b).wait()
            pltpu.sync_copy(
                rows_b,
                out_hbm.at[pl.ds(base + j * _CHUNK, _CHUNK),
                           pl.ds(0, EMBED_DIM)])
            pltpu.async_copy(table_hbm.at[idx_v.at[j + 2]], rows_b, sem_b)

        step(rows0, sem0, j0)
        step(rows1, sem1, j0 + 1)
        return carry

    lax.fori_loop(0, _N_CHUNKS // 2 - 1, body, 0)

    j_last = _N_CHUNKS - 2
    pltpu.make_async_copy(
        table_hbm.at[idx_v.at[j_last]], rows0, sem0).wait()
    pltpu.sync_copy(
        rows0,
        out_hbm.at[pl.ds(base + j_last * _CHUNK, _CHUNK),
                   pl.ds(0, EMBED_DIM)])
    pltpu.make_async_copy(
        table_hbm.at[idx_v.at[j_last + 1]], rows1, sem1).wait()
    pltpu.sync_copy(
        rows1,
        out_hbm.at[pl.ds(base + (j_last + 1) * _CHUNK, _CHUNK),
                   pl.ds(0, EMBED_DIM)])


def _build():
    mesh = plsc.VectorSubcoreMesh(core_axis_name="c", subcore_axis_name="s")
    return functools.partial(
        pl.kernel,
        mesh=mesh,
        out_type=jax.ShapeDtypeStruct((_N_IDX, 2 * EMBED_DIM), jnp.float32),
        scratch_types=[
            pltpu.VMEM((_N_CHUNKS, _CHUNK), jnp.int32),
            pltpu.VMEM((_CHUNK, EMBED_DIM), jnp.float32),
            pltpu.VMEM((_CHUNK, EMBED_DIM), jnp.float32),
            pltpu.SemaphoreType.DMA,
            pltpu.SemaphoreType.DMA,
        ],
        compiler_params=pltpu.CompilerParams(
            use_tc_tiling_on_sc=False, needs_layout_passes=False),
    )(_emb_kernel)


def kernel(x, embedding_weight):
    idx = x.reshape(_NW, _N_CHUNKS, _CHUNK).astype(jnp.int32)
    table_lin = _prep_tc(embedding_weight.T).reshape(VOCAB_SIZE, EMBED_DIM)
    p = _build()(idx, table_lin)
    return p.reshape(BATCH, HIST_LEN, 2 * EMBED_DIM)[:, :, :EMBED_DIM]


# speedup vs baseline: 1.2229x; 1.2229x over previous
import functools

import jax
import jax.numpy as jnp
from jax import lax
from jax.experimental import pallas as pl
from jax.experimental.pallas import tpu as pltpu
from jax.experimental.pallas import tpu_sc as plsc

VOCAB_SIZE = 1000000
EMBED_DIM = 64
BATCH = 4096
HIST_LEN = 200

_NC = 2
_NS = 16
_NW = _NC * _NS
_CHUNK = 128
_N_IDX = BATCH * HIST_LEN
_PER_W = _N_IDX // _NW
_N_CHUNKS = _PER_W // _CHUNK

_BLKV = 8192
_GRID = (VOCAB_SIZE + _BLKV - 1) // _BLKV


def _prep_body(t_ref, o_ref):
    a = t_ref[...].T.reshape(_BLKV // 2, 2, EMBED_DIM)
    o_ref[...] = jnp.concatenate([a[:, 0, :], a[:, 1, :]], axis=-1)


def _prep_tc(table_t):
    return pl.pallas_call(
        _prep_body,
        grid=(_GRID,),
        in_specs=[pl.BlockSpec((EMBED_DIM, _BLKV), lambda i: (0, i))],
        out_specs=pl.BlockSpec((_BLKV // 2, 2 * EMBED_DIM),
                               lambda i: (i, 0)),
        out_shape=jax.ShapeDtypeStruct(
            (VOCAB_SIZE // 2, 2 * EMBED_DIM), jnp.float32),
    )(table_t)


def _emb_kernel(idx_hbm, table_hbm, out_hbm, idx_v, rows0, rows1, sem0, sem1):
    wid = lax.axis_index("s") * _NC + lax.axis_index("c")
    base = wid * _PER_W

    pltpu.sync_copy(idx_hbm.at[wid], idx_v)

    pltpu.async_copy(table_hbm.at[idx_v.at[0]], rows0, sem0)
    pltpu.async_copy(table_hbm.at[idx_v.at[1]], rows1, sem1)

    def body(t, carry):
        j0 = 2 * t

        def step(rows_b, sem_b, j):
            pltpu.make_async_copy(
                table_hbm.at[idx_v.at[j]], rows_b, sem_b).wait()
            pltpu.sync_copy(
                rows_b,
                out_hbm.at[pl.ds(base + j * _CHUNK, _CHUNK),
                           pl.ds(0, EMBED_DIM)])
            pltpu.async_copy(table_hbm.at[idx_v.at[j + 2]], rows_b, sem_b)

        step(rows0, sem0, j0)
        step(rows1, sem1, j0 + 1)
        return carry

    lax.fori_loop(0, _N_CHUNKS // 2 - 1, body, 0)

    j_last = _N_CHUNKS - 2
    pltpu.make_async_copy(
        table_hbm.at[idx_v.at[j_last]], rows0, sem0).wait()
    pltpu.sync_copy(
        rows0,
        out_hbm.at[pl.ds(base + j_last * _CHUNK, _CHUNK),
                   pl.ds(0, EMBED_DIM)])
    pltpu.make_async_copy(
        table_hbm.at[idx_v.at[j_last + 1]], rows1, sem1).wait()
    pltpu.sync_copy(
        rows1,
        out_hbm.at[pl.ds(base + (j_last + 1) * _CHUNK, _CHUNK),
                   pl.ds(0, EMBED_DIM)])


def _build():
    mesh = plsc.VectorSubcoreMesh(core_axis_name="c", subcore_axis_name="s")
    return functools.partial(
        pl.kernel,
        mesh=mesh,
        out_type=jax.ShapeDtypeStruct((_N_IDX, 2 * EMBED_DIM), jnp.float32),
        scratch_types=[
            pltpu.VMEM((_N_CHUNKS, _CHUNK), jnp.int32),
            pltpu.VMEM((_CHUNK, EMBED_DIM), jnp.float32),
            pltpu.VMEM((_CHUNK, EMBED_DIM), jnp.float32),
            pltpu.SemaphoreType.DMA,
            pltpu.SemaphoreType.DMA,
        ],
        compiler_params=pltpu.CompilerParams(
            use_tc_tiling_on_sc=False, needs_layout_passes=False),
    )(_emb_kernel)


def kernel(x, embedding_weight):
    idx = x.reshape(_NW, _N_CHUNKS, _CHUNK).astype(jnp.int32)
    table_lin = _prep_tc(embedding_weight.T).reshape(VOCAB_SIZE, EMBED_DIM)
    p = _build()(idx, table_lin)
    return p.reshape(BATCH, HIST_LEN, 2 * EMBED_DIM)[:, :, :EMBED_DIM]


# speedup vs baseline: 1.2276x; 1.0039x over previous
import functools

import jax
import jax.numpy as jnp
from jax import lax
from jax.experimental import pallas as pl
from jax.experimental.pallas import tpu as pltpu
from jax.experimental.pallas import tpu_sc as plsc

VOCAB_SIZE = 1000000
EMBED_DIM = 64
BATCH = 4096
HIST_LEN = 200

_NC = 2
_NS = 16
_NW = _NC * _NS
_CHUNK = 128
_N_IDX = BATCH * HIST_LEN
_PER_W = _N_IDX // _NW
_N_CHUNKS = _PER_W // _CHUNK

_BLKV = 16384
_GRID = (VOCAB_SIZE + _BLKV - 1) // _BLKV


def _prep_body(t_ref, o_ref):
    a = t_ref[...].T.reshape(_BLKV // 2, 2, EMBED_DIM)
    o_ref[...] = jnp.concatenate([a[:, 0, :], a[:, 1, :]], axis=-1)


def _prep_tc(table_t):
    return pl.pallas_call(
        _prep_body,
        grid=(_GRID,),
        in_specs=[pl.BlockSpec((EMBED_DIM, _BLKV), lambda i: (0, i))],
        out_specs=pl.BlockSpec((_BLKV // 2, 2 * EMBED_DIM),
                               lambda i: (i, 0)),
        out_shape=jax.ShapeDtypeStruct(
            (VOCAB_SIZE // 2, 2 * EMBED_DIM), jnp.float32),
    )(table_t)


def _emb_kernel(idx_hbm, table_hbm, out_hbm, idx_v, rows0, rows1, sem0, sem1):
    wid = lax.axis_index("s") * _NC + lax.axis_index("c")
    base = wid * _PER_W

    pltpu.sync_copy(idx_hbm.at[wid], idx_v)

    pltpu.async_copy(table_hbm.at[idx_v.at[0]], rows0, sem0)
    pltpu.async_copy(table_hbm.at[idx_v.at[1]], rows1, sem1)

    def body(t, carry):
        j0 = 2 * t

        def step(rows_b, sem_b, j):
            pltpu.make_async_copy(
                table_hbm.at[idx_v.at[j]], rows_b, sem_b).wait()
            pltpu.sync_copy(
                rows_b,
                out_hbm.at[pl.ds(base + j * _CHUNK, _CHUNK),
                           pl.ds(0, EMBED_DIM)])
            pltpu.async_copy(table_hbm.at[idx_v.at[j + 2]], rows_b, sem_b)

        step(rows0, sem0, j0)
        step(rows1, sem1, j0 + 1)
        return carry

    lax.fori_loop(0, _N_CHUNKS // 2 - 1, body, 0)

    j_last = _N_CHUNKS - 2
    pltpu.make_async_copy(
        table_hbm.at[idx_v.at[j_last]], rows0, sem0).wait()
    pltpu.sync_copy(
        rows0,
        out_hbm.at[pl.ds(base + j_last * _CHUNK, _CHUNK),
                   pl.ds(0, EMBED_DIM)])
    pltpu.make_async_copy(
        table_hbm.at[idx_v.at[j_last + 1]], rows1, sem1).wait()
    pltpu.sync_copy(
        rows1,
        out_hbm.at[pl.ds(base + (j_last + 1) * _CHUNK, _CHUNK),
                   pl.ds(0, EMBED_DIM)])


def _build():
    mesh = plsc.VectorSubcoreMesh(core_axis_name="c", subcore_axis_name="s")
    return functools.partial(
        pl.kernel,
        mesh=mesh,
        out_type=jax.ShapeDtypeStruct((_N_IDX, 2 * EMBED_DIM), jnp.float32),
        scratch_types=[
            pltpu.VMEM((_N_CHUNKS, _CHUNK), jnp.int32),
            pltpu.VMEM((_CHUNK, EMBED_DIM), jnp.float32),
            pltpu.VMEM((_CHUNK, EMBED_DIM), jnp.float32),
            pltpu.SemaphoreType.DMA,
            pltpu.SemaphoreType.DMA,
        ],
        compiler_params=pltpu.CompilerParams(
            use_tc_tiling_on_sc=False, needs_layout_passes=False),
    )(_emb_kernel)


def kernel(x, embedding_weight):
    idx = x.reshape(_NW, _N_CHUNKS, _CHUNK).astype(jnp.int32)
    table_lin = _prep_tc(embedding_weight.T).reshape(VOCAB_SIZE, EMBED_DIM)
    p = _build()(idx, table_lin)
    return p.reshape(BATCH, HIST_LEN, 2 * EMBED_DIM)[:, :, :EMBED_DIM]
